# baseline (speedup 1.0000x reference)
import jax
import jax.numpy as jnp
from jax.experimental import pallas as pl
from jax.experimental.pallas import tpu as pltpu

_NC = 256
_EPS = 1e-8


def _fused_kernel(x_ref, am_ref, out_ref):
    x = x_ref[...]
    am2 = am_ref[...]
    dots = jax.lax.dot_general(
        am2, x, (((1,), (0,)), ((), ())),
        preferred_element_type=jnp.float32)
    xn = jnp.sqrt(jnp.sum(x * x, axis=0, keepdims=True))
    yn = jnp.sqrt(jnp.sum(am2 * am2, axis=1, keepdims=True))
    den = jnp.maximum(xn * yn, _EPS)
    sims = dots / den
    out_ref[...] = (sims[1:2, :] > sims[0:1, :]).astype(jnp.int32)


def kernel(enc_hvs, am):
    n, d = enc_hvs.shape
    xt = enc_hvs.T
    am = am.astype(jnp.float32)
    out = pl.pallas_call(
        _fused_kernel,
        grid=(n // _NC,),
        in_specs=[
            pl.BlockSpec((d, _NC), lambda i: (0, i)),
            pl.BlockSpec((2, d), lambda i: (0, 0)),
        ],
        out_specs=pl.BlockSpec((1, _NC), lambda i: (0, i)),
        out_shape=jax.ShapeDtypeStruct((1, n), jnp.int32),
        compiler_params=pltpu.CompilerParams(
            dimension_semantics=("arbitrary",),
            vmem_limit_bytes=60 * 1024 * 1024,
        ),
    )(xt, am)
    return out.reshape(n)


# speedup vs baseline: 1.0033x; 1.0033x over previous
import jax
import jax.numpy as jnp
from jax.experimental import pallas as pl
from jax.experimental.pallas import tpu as pltpu

_NC = 512
_EPS = 1e-8


def _fused_kernel(x_ref, am_ref, out_ref):
    x = x_ref[...]
    am2 = am_ref[...]
    dots = jax.lax.dot_general(
        am2, x, (((1,), (0,)), ((), ())),
        preferred_element_type=jnp.float32)
    xn = jnp.sqrt(jnp.sum(x * x, axis=0, keepdims=True))
    yn = jnp.sqrt(jnp.sum(am2 * am2, axis=1, keepdims=True))
    den = jnp.maximum(xn * yn, _EPS)
    sims = dots / den
    out_ref[...] = (sims[1:2, :] > sims[0:1, :]).astype(jnp.int32)


def kernel(enc_hvs, am):
    n, d = enc_hvs.shape
    xt = enc_hvs.T
    am = am.astype(jnp.float32)
    out = pl.pallas_call(
        _fused_kernel,
        grid=(n // _NC,),
        in_specs=[
            pl.BlockSpec((d, _NC), lambda i: (0, i)),
            pl.BlockSpec((2, d), lambda i: (0, 0)),
        ],
        out_specs=pl.BlockSpec((1, _NC), lambda i: (0, i)),
        out_shape=jax.ShapeDtypeStruct((1, n), jnp.int32),
        compiler_params=pltpu.CompilerParams(
            dimension_semantics=("arbitrary",),
            vmem_limit_bytes=60 * 1024 * 1024,
        ),
    )(xt, am)
    return out.reshape(n)
